# baseline (device time: 64429 ns/iter reference)
import jax
import jax.numpy as jnp
from jax import lax
from jax.experimental import pallas as pl
from jax.experimental.pallas import tpu as pltpu

N_DEV = 4
SQ = 1024
SKV_LOC = 1024
HQ = 8
DH = 128
D = 1024
SCALE = 0.08838834764831843

F32 = jnp.float32
BF16 = jnp.bfloat16
MESH = pl.DeviceIdType.MESH


def kernel(x, Wq, K_ext, V_ext, Wo):
    def body(x_ref, wq_ref, k_ref, v_ref, wo_ref, out_ref,
             p0_ref, d0l_ref, edge_ref, glob_ref, acc_ref, lt_ref,
             madd_ref, kb_ref, vb_ref, kv_sems, sp0, rp0, s_small, r_small):
        my = lax.axis_index("i")

        barrier = pltpu.get_barrier_semaphore()
        for t in range(N_DEV):
            @pl.when(my != t)
            def _():
                pl.semaphore_signal(barrier, inc=1, device_id=(t,),
                                    device_id_type=MESH)
        pl.semaphore_wait(barrier, N_DEV - 1)

        def kv_dma(h):
            return (pltpu.make_async_copy(k_ref.at[0, :, h, :], kb_ref.at[h],
                                          kv_sems.at[0, h]),
                    pltpu.make_async_copy(v_ref.at[0, :, h, :], vb_ref.at[h],
                                          kv_sems.at[1, h]))

        for h in range(HQ):
            for d in kv_dma(h):
                d.start()

        def kv_wait(h):
            for d in kv_dma(h):
                d.wait()

        def copy(src, dst, ssem, rsem, dev):
            return pltpu.make_async_remote_copy(
                src_ref=src, dst_ref=dst, send_sem=ssem, recv_sem=rsem,
                device_id=(dev,), device_id_type=MESH)

        def send_small(buf, base, targets, ridx):
            for j, t in enumerate(targets):
                copy(buf, buf, s_small.at[base + j], r_small.at[ridx],
                     t).start()

        def wait_small(buf, ridx):
            copy(buf, buf, s_small.at[0], r_small.at[ridx], 0).wait_recv()

        def q_head(rows, h):
            return jnp.dot(x_ref[0, rows, :], wq_ref[:, pl.ds(h * DH, DH)],
                           preferred_element_type=F32) * SCALE

        def glob_partial(h):
            k_h, v_h = kb_ref[h], vb_ref[h]
            qg = q_head(pl.ds(0, 32), h)
            sg = lax.dot_general(qg, k_h, (((1,), (1,)), ((), ())),
                                 preferred_element_type=F32)
            wg = jnp.exp(sg)
            return (jnp.dot(wg, v_h, preferred_element_type=F32),
                    jnp.sum(wg, axis=1, keepdims=True))

        def zero_mid():
            out_ref[0, pl.ds(32, 864), :] = jnp.zeros((864, D), F32)

        def wo_acc(h):
            hs = pl.ds(h * DH, DH)
            out_ref[0, pl.ds(32, 480), :] = (
                out_ref[0, pl.ds(32, 480), :]
                + jnp.dot(p0_ref[h, 0, pl.ds(32, 480)].astype(F32),
                          wo_ref[hs, :], preferred_element_type=F32))
            out_ref[0, pl.ds(512, 384), :] = (
                out_ref[0, pl.ds(512, 384), :]
                + jnp.dot(p0_ref[h, 1, pl.ds(0, 384)].astype(F32),
                          wo_ref[hs, :], preferred_element_type=F32))

        STARTS = (0, 128, 384, 512)
        @pl.when(my == 0)
        def _dev0():
            for b in range(4):
                qi = lax.broadcasted_iota(jnp.int32, (256, 512), 0) + 256 * b
                kj = lax.broadcasted_iota(jnp.int32, (256, 512), 1) + STARTS[b]
                m = jnp.abs(qi - kj) <= 128
                if b == 0:
                    m = m | (kj < 32) | (qi < 32)
                madd_ref[b] = jnp.where(m, 0.0, -1e9)
            for h in range(HQ):
                kv_wait(h)
                q_h = q_head(slice(None), h)
                for b in range(4):
                    st = STARTS[b]
                    qb = q_h[b * 256:(b + 1) * 256]
                    sb = lax.dot_general(qb, kb_ref[h, pl.ds(st, 512)],
                                         (((1,), (1,)), ((), ())),
                                         preferred_element_type=F32)
                    wb = jnp.exp(sb + madd_ref[b])
                    lb = jnp.sum(wb, axis=1, keepdims=True)
                    ctxb = jnp.dot(wb, vb_ref[h, pl.ds(st, 512)],
                                   preferred_element_type=F32)
                    if b > 0:
                        st_g = lax.dot_general(qb, kb_ref[h, pl.ds(0, 32)],
                                               (((1,), (1,)), ((), ())),
                                               preferred_element_type=F32)
                        wt = jnp.exp(st_g)
                        lb = lb + jnp.sum(wt, axis=1, keepdims=True)
                        ctxb = ctxb + jnp.dot(wt, vb_ref[h, pl.ds(0, 32)],
                                              preferred_element_type=F32)
                    cn = (ctxb / lb).astype(BF16)
                    p0_ref[h, b // 2, pl.ds((b % 2) * 256, 256)] = cn
                    if b == 0:
                        sc = lax.dot_general(q_h[0:32],
                                             kb_ref[h, pl.ds(512, 512)],
                                             (((1,), (1,)), ((), ())),
                                             preferred_element_type=F32)
                        wc = jnp.exp(sc)
                        ctx32 = ctxb[0:32] + jnp.dot(
                            wc, vb_ref[h, pl.ds(512, 512)],
                            preferred_element_type=F32)
                        l32 = lb[0:32] + jnp.sum(wc, axis=1, keepdims=True)
                        p0_ref[h, 0, pl.ds(0, 32)] = ctx32.astype(BF16)
                        d0l_ref[pl.ds(0, 32), pl.ds(h, 1)] = l32
                    if b == 1:
                        for j, t in enumerate((1, 3)):
                            copy(p0_ref.at[h, 0], p0_ref.at[h, 0],
                                 sp0.at[2 * j, h], rp0.at[0, h], t).start()
                    if b == 3:
                        p0_ref[h, 1, pl.ds(384, 128)] = (
                            ctxb[128:256].astype(BF16))
                        d0l_ref[pl.ds(32, 128), pl.ds(h, 1)] = lb[128:256]
                        for j, t in enumerate((1, 3)):
                            copy(p0_ref.at[h, 1], p0_ref.at[h, 1],
                                 sp0.at[2 * j + 1, h], rp0.at[1, h],
                                 t).start()
            send_small(d0l_ref, 0, (1, 2, 3), 0)
            zero_mid()
            for h in range(HQ):
                wo_acc(h)
            for ridx in (1, 2, 3, 4):
                wait_small(edge_ref if ridx == 1 else
                           glob_ref.at[ridx - 2], ridx)

        @pl.when(my == 1)
        def _dev1():
            qi = lax.broadcasted_iota(jnp.int32, (128, SKV_LOC), 0) + 896
            kj = lax.broadcasted_iota(jnp.int32, (128, SKV_LOC), 1) + SKV_LOC
            madd_e = jnp.where(jnp.abs(qi - kj) <= 128, 0.0, -1e9)
            for h in range(HQ):
                kv_wait(h)
                k_h, v_h = kb_ref[h], vb_ref[h]
                ctx_g, l_g = glob_partial(h)
                glob_ref[0, h] = ctx_g.astype(BF16)
                glob_ref[0, 8, :, pl.ds(h, 1)] = l_g.astype(BF16)
                qe = q_head(pl.ds(896, 128), h)
                se = lax.dot_general(qe, k_h, (((1,), (1,)), ((), ())),
                                     preferred_element_type=F32)
                we = jnp.exp(se + madd_e)
                edge_ref[h] = jnp.dot(we, v_h,
                                      preferred_element_type=F32).astype(BF16)
                edge_ref[8, :, pl.ds(h, 1)] = jnp.sum(
                    we, axis=1, keepdims=True).astype(BF16)
            send_small(edge_ref, 0, (0, 2, 3), 1)
            send_small(glob_ref.at[0], 3, (0, 2, 3), 2)
            zero_mid()
            for h in range(HQ):
                copy(p0_ref.at[h, 0], p0_ref.at[h, 0], sp0.at[1, h],
                     rp0.at[0, h], 0).wait_recv()
                copy(p0_ref.at[h, 0], p0_ref.at[h, 0], sp0.at[0, h],
                     rp0.at[0, h], 2).start()
                copy(p0_ref.at[h, 1], p0_ref.at[h, 1], sp0.at[1, h],
                     rp0.at[1, h], 0).wait_recv()
                wo_acc(h)
            wait_small(d0l_ref, 0)
            wait_small(glob_ref.at[1], 3)
            wait_small(glob_ref.at[2], 4)

        @pl.when(my == 2)
        def _dev2():
            for h in range(HQ):
                kv_wait(h)
                ctx_g, l_g = glob_partial(h)
                glob_ref[1, h] = ctx_g.astype(BF16)
                glob_ref[1, 8, :, pl.ds(h, 1)] = l_g.astype(BF16)
            send_small(glob_ref.at[1], 0, (0, 1, 3), 3)
            zero_mid()
            for h in range(HQ):
                copy(p0_ref.at[h, 0], p0_ref.at[h, 0], sp0.at[1, h],
                     rp0.at[0, h], 0).wait_recv()
                copy(p0_ref.at[h, 1], p0_ref.at[h, 1], sp0.at[1, h],
                     rp0.at[1, h], 0).wait_recv()
                wo_acc(h)
            wait_small(d0l_ref, 0)
            wait_small(edge_ref, 1)
            wait_small(glob_ref.at[0], 2)
            wait_small(glob_ref.at[2], 4)

        @pl.when(my == 3)
        def _dev3():
            for h in range(HQ):
                kv_wait(h)
                ctx_g, l_g = glob_partial(h)
                glob_ref[2, h] = ctx_g.astype(BF16)
                glob_ref[2, 8, :, pl.ds(h, 1)] = l_g.astype(BF16)
            send_small(glob_ref.at[2], 0, (0, 1, 2), 4)
            zero_mid()
            for h in range(HQ):
                copy(p0_ref.at[h, 1], p0_ref.at[h, 1], sp0.at[1, h],
                     rp0.at[1, h], 0).wait_recv()
                copy(p0_ref.at[h, 1], p0_ref.at[h, 1], sp0.at[0, h],
                     rp0.at[1, h], 2).start()
                copy(p0_ref.at[h, 0], p0_ref.at[h, 0], sp0.at[1, h],
                     rp0.at[0, h], 0).wait_recv()
                wo_acc(h)
            wait_small(d0l_ref, 0)
            wait_small(edge_ref, 1)
            wait_small(glob_ref.at[0], 2)
            wait_small(glob_ref.at[1], 3)

        for h in range(HQ):
            hs = pl.ds(h * DH, DH)
            acc_ref[pl.ds(0, 32), hs] = (p0_ref[h, 0, pl.ds(0, 32)].astype(F32)
                                         + glob_ref[0, h].astype(F32)
                                         + glob_ref[1, h].astype(F32)
                                         + glob_ref[2, h].astype(F32))
            acc_ref[pl.ds(32, 128), hs] = (
                p0_ref[h, 1, pl.ds(384, 128)].astype(F32)
                + edge_ref[h].astype(F32))
        lt_ref[pl.ds(0, 32), :] = (d0l_ref[pl.ds(0, 32), :]
                                   + glob_ref[0, 8, :, 0:8].astype(F32)
                                   + glob_ref[1, 8, :, 0:8].astype(F32)
                                   + glob_ref[2, 8, :, 0:8].astype(F32))
        lt_ref[pl.ds(32, 128), :] = (d0l_ref[pl.ds(32, 128), :]
                                     + edge_ref[8, :, 0:8].astype(F32))
        for h in range(HQ):
            hs = pl.ds(h * DH, DH)
            acc_ref[pl.ds(0, 32), hs] = (acc_ref[pl.ds(0, 32), hs]
                                         / lt_ref[pl.ds(0, 32), pl.ds(h, 1)])
            acc_ref[pl.ds(32, 128), hs] = (acc_ref[pl.ds(32, 128), hs]
                                           / lt_ref[pl.ds(32, 128),
                                                    pl.ds(h, 1)])
        out_ref[0, pl.ds(0, 32), :] = jnp.dot(
            acc_ref[pl.ds(0, 32), :], wo_ref[...],
            preferred_element_type=F32)
        out_ref[0, pl.ds(896, 128), :] = jnp.dot(
            acc_ref[pl.ds(32, 128), :], wo_ref[...],
            preferred_element_type=F32)

        @pl.when(my == 0)
        def _():
            for h in range(HQ):
                for j in range(2):
                    copy(p0_ref.at[h, 0], p0_ref.at[h, 0], sp0.at[2 * j, h],
                         rp0.at[0, h], 1).wait_send()
                    copy(p0_ref.at[h, 1], p0_ref.at[h, 1],
                         sp0.at[2 * j + 1, h], rp0.at[1, h], 1).wait_send()
            for j in range(3):
                copy(d0l_ref, d0l_ref, s_small.at[j], r_small.at[0],
                     1).wait_send()

        @pl.when(my == 1)
        def _():
            for h in range(HQ):
                copy(p0_ref.at[h, 0], p0_ref.at[h, 0], sp0.at[0, h],
                     rp0.at[0, h], 2).wait_send()
            for j in range(3):
                copy(edge_ref, edge_ref, s_small.at[j], r_small.at[1],
                     0).wait_send()
                copy(glob_ref.at[0], glob_ref.at[0], s_small.at[3 + j],
                     r_small.at[2], 0).wait_send()

        @pl.when(my == 2)
        def _():
            for j in range(3):
                copy(glob_ref.at[1], glob_ref.at[1], s_small.at[j],
                     r_small.at[3], 0).wait_send()

        @pl.when(my == 3)
        def _():
            for h in range(HQ):
                copy(p0_ref.at[h, 1], p0_ref.at[h, 1], sp0.at[0, h],
                     rp0.at[1, h], 2).wait_send()
            for j in range(3):
                copy(glob_ref.at[2], glob_ref.at[2], s_small.at[j],
                     r_small.at[4], 0).wait_send()

    return pl.pallas_call(
        body,
        out_shape=jax.ShapeDtypeStruct((1, SQ, D), F32),
        in_specs=[pl.BlockSpec(memory_space=pltpu.MemorySpace.VMEM),
                  pl.BlockSpec(memory_space=pltpu.MemorySpace.VMEM),
                  pl.BlockSpec(memory_space=pltpu.MemorySpace.HBM),
                  pl.BlockSpec(memory_space=pltpu.MemorySpace.HBM),
                  pl.BlockSpec(memory_space=pltpu.MemorySpace.VMEM)],
        out_specs=pl.BlockSpec(memory_space=pltpu.MemorySpace.VMEM),
        scratch_shapes=[
            pltpu.VMEM((HQ, 2, 512, DH), BF16),
            pltpu.VMEM((160, HQ), F32),
            pltpu.VMEM((HQ + 1, 128, DH), BF16),
            pltpu.VMEM((3, HQ + 1, 32, DH), BF16),
            pltpu.VMEM((160, D), F32),
            pltpu.VMEM((160, HQ), F32),
            pltpu.VMEM((4, 256, 512), F32),
            pltpu.VMEM((HQ, SKV_LOC, DH), F32),
            pltpu.VMEM((HQ, SKV_LOC, DH), F32),
            pltpu.SemaphoreType.DMA((2, HQ)),
            pltpu.SemaphoreType.DMA((4, HQ)),
            pltpu.SemaphoreType.DMA((2, HQ)),
            pltpu.SemaphoreType.DMA((6,)),
            pltpu.SemaphoreType.DMA((5,)),
        ],
        compiler_params=pltpu.CompilerParams(collective_id=0),
    )(x, Wq, K_ext, V_ext, Wo)


# device time: 55720 ns/iter; 1.1563x vs baseline; 1.1563x over previous
import jax
import jax.numpy as jnp
from jax import lax
from jax.experimental import pallas as pl
from jax.experimental.pallas import tpu as pltpu

N_DEV = 4
SQ = 1024
SKV_LOC = 1024
HQ = 8
DH = 128
D = 1024
SCALE = 0.08838834764831843

F32 = jnp.float32
BF16 = jnp.bfloat16
MESH = pl.DeviceIdType.MESH


def kernel(x, Wq, K_ext, V_ext, Wo):
    K = jnp.transpose(K_ext[0], (1, 0, 2)).astype(BF16)
    V = jnp.transpose(V_ext[0], (1, 0, 2))

    def body(x_ref, wq_ref, k_ref, v_ref, wo_ref, out_ref,
             p0_ref, d0l_ref, edge_ref, glob_ref, acc_ref, lt_ref,
             madd_ref, sp0, rp0, s_small, r_small):
        my = lax.axis_index("i")

        barrier = pltpu.get_barrier_semaphore()
        for t in range(N_DEV):
            @pl.when(my != t)
            def _():
                pl.semaphore_signal(barrier, inc=1, device_id=(t,),
                                    device_id_type=MESH)
        pl.semaphore_wait(barrier, N_DEV - 1)

        def copy(src, dst, ssem, rsem, dev):
            return pltpu.make_async_remote_copy(
                src_ref=src, dst_ref=dst, send_sem=ssem, recv_sem=rsem,
                device_id=(dev,), device_id_type=MESH)

        def send_small(buf, base, targets, ridx):
            for j, t in enumerate(targets):
                copy(buf, buf, s_small.at[base + j], r_small.at[ridx],
                     t).start()

        def wait_small(buf, ridx):
            copy(buf, buf, s_small.at[0], r_small.at[ridx], 0).wait_recv()

        def q_head(rows, h):
            return (jnp.dot(x_ref[0, rows, :], wq_ref[:, pl.ds(h * DH, DH)],
                            preferred_element_type=F32)
                    * SCALE).astype(BF16)

        def glob_partial(h):
            k_h, v_h = k_ref[h], v_ref[h]
            qg = q_head(pl.ds(0, 32), h)
            sg = lax.dot_general(qg, k_h, (((1,), (1,)), ((), ())),
                                 preferred_element_type=F32)
            wg = jnp.exp(sg)
            return (jnp.dot(wg, v_h, preferred_element_type=F32),
                    jnp.sum(wg, axis=1, keepdims=True))

        def zero_mid():
            out_ref[0, pl.ds(32, 864), :] = jnp.zeros((864, D), F32)

        def wo_acc(h):
            hs = pl.ds(h * DH, DH)
            out_ref[0, pl.ds(32, 480), :] = (
                out_ref[0, pl.ds(32, 480), :]
                + jnp.dot(p0_ref[h, 0, pl.ds(32, 480)].astype(F32),
                          wo_ref[hs, :], preferred_element_type=F32))
            out_ref[0, pl.ds(512, 384), :] = (
                out_ref[0, pl.ds(512, 384), :]
                + jnp.dot(p0_ref[h, 1, pl.ds(0, 384)].astype(F32),
                          wo_ref[hs, :], preferred_element_type=F32))

        STARTS = (0, 128, 384, 512)
        @pl.when(my == 0)
        def _dev0():
            for b in range(4):
                qi = lax.broadcasted_iota(jnp.int32, (256, 512), 0) + 256 * b
                kj = lax.broadcasted_iota(jnp.int32, (256, 512), 1) + STARTS[b]
                m = jnp.abs(qi - kj) <= 128
                if b == 0:
                    m = m | (kj < 32) | (qi < 32)
                madd_ref[b] = jnp.where(m, 0.0, -1e9)
            for h in range(HQ):
                q_h = q_head(slice(None), h)
                for b in range(4):
                    st = STARTS[b]
                    qb = q_h[b * 256:(b + 1) * 256]
                    sb = lax.dot_general(qb, k_ref[h, pl.ds(st, 512)],
                                         (((1,), (1,)), ((), ())),
                                         preferred_element_type=F32)
                    wb = jnp.exp(sb + madd_ref[b])
                    lb = jnp.sum(wb, axis=1, keepdims=True)
                    ctxb = jnp.dot(wb, v_ref[h, pl.ds(st, 512)],
                                   preferred_element_type=F32)
                    if b > 0:
                        st_g = lax.dot_general(qb, k_ref[h, pl.ds(0, 32)],
                                               (((1,), (1,)), ((), ())),
                                               preferred_element_type=F32)
                        wt = jnp.exp(st_g)
                        lb = lb + jnp.sum(wt, axis=1, keepdims=True)
                        ctxb = ctxb + jnp.dot(wt, v_ref[h, pl.ds(0, 32)],
                                              preferred_element_type=F32)
                    cn = (ctxb / lb).astype(BF16)
                    p0_ref[h, b // 2, pl.ds((b % 2) * 256, 256)] = cn
                    if b == 0:
                        sc = lax.dot_general(q_h[0:32],
                                             k_ref[h, pl.ds(512, 512)],
                                             (((1,), (1,)), ((), ())),
                                             preferred_element_type=F32)
                        wc = jnp.exp(sc)
                        ctx32 = ctxb[0:32] + jnp.dot(
                            wc, v_ref[h, pl.ds(512, 512)],
                            preferred_element_type=F32)
                        l32 = lb[0:32] + jnp.sum(wc, axis=1, keepdims=True)
                        p0_ref[h, 0, pl.ds(0, 32)] = ctx32.astype(BF16)
                        d0l_ref[pl.ds(0, 32), pl.ds(h, 1)] = l32
                    if b == 1:
                        for j, t in enumerate((1, 3)):
                            copy(p0_ref.at[h, 0], p0_ref.at[h, 0],
                                 sp0.at[2 * j, h], rp0.at[0, h], t).start()
                    if b == 3:
                        p0_ref[h, 1, pl.ds(384, 128)] = (
                            ctxb[128:256].astype(BF16))
                        d0l_ref[pl.ds(32, 128), pl.ds(h, 1)] = lb[128:256]
                        for j, t in enumerate((1, 3)):
                            copy(p0_ref.at[h, 1], p0_ref.at[h, 1],
                                 sp0.at[2 * j + 1, h], rp0.at[1, h],
                                 t).start()
            send_small(d0l_ref, 0, (1, 2, 3), 0)
            zero_mid()
            for h in range(HQ):
                wo_acc(h)
            for ridx in (1, 2, 3, 4):
                wait_small(edge_ref if ridx == 1 else
                           glob_ref.at[ridx - 2], ridx)

        @pl.when(my == 1)
        def _dev1():
            qi = lax.broadcasted_iota(jnp.int32, (128, SKV_LOC), 0) + 896
            kj = lax.broadcasted_iota(jnp.int32, (128, SKV_LOC), 1) + SKV_LOC
            madd_e = jnp.where(jnp.abs(qi - kj) <= 128, 0.0, -1e9)
            for h in range(HQ):
                k_h, v_h = k_ref[h], v_ref[h]
                ctx_g, l_g = glob_partial(h)
                glob_ref[0, h] = ctx_g.astype(BF16)
                glob_ref[0, 8, :, pl.ds(h, 1)] = l_g.astype(BF16)
                qe = q_head(pl.ds(896, 128), h)
                se = lax.dot_general(qe, k_h, (((1,), (1,)), ((), ())),
                                     preferred_element_type=F32)
                we = jnp.exp(se + madd_e)
                edge_ref[h] = jnp.dot(we, v_h,
                                      preferred_element_type=F32).astype(BF16)
                edge_ref[8, :, pl.ds(h, 1)] = jnp.sum(
                    we, axis=1, keepdims=True).astype(BF16)
            send_small(edge_ref, 0, (0, 2, 3), 1)
            send_small(glob_ref.at[0], 3, (0, 2, 3), 2)
            zero_mid()
            for h in range(HQ):
                copy(p0_ref.at[h, 0], p0_ref.at[h, 0], sp0.at[1, h],
                     rp0.at[0, h], 0).wait_recv()
                copy(p0_ref.at[h, 0], p0_ref.at[h, 0], sp0.at[0, h],
                     rp0.at[0, h], 2).start()
                copy(p0_ref.at[h, 1], p0_ref.at[h, 1], sp0.at[1, h],
                     rp0.at[1, h], 0).wait_recv()
                wo_acc(h)
            wait_small(d0l_ref, 0)
            wait_small(glob_ref.at[1], 3)
            wait_small(glob_ref.at[2], 4)

        @pl.when(my == 2)
        def _dev2():
            for h in range(HQ):
                ctx_g, l_g = glob_partial(h)
                glob_ref[1, h] = ctx_g.astype(BF16)
                glob_ref[1, 8, :, pl.ds(h, 1)] = l_g.astype(BF16)
            send_small(glob_ref.at[1], 0, (0, 1, 3), 3)
            zero_mid()
            for h in range(HQ):
                copy(p0_ref.at[h, 0], p0_ref.at[h, 0], sp0.at[1, h],
                     rp0.at[0, h], 0).wait_recv()
                copy(p0_ref.at[h, 1], p0_ref.at[h, 1], sp0.at[1, h],
                     rp0.at[1, h], 0).wait_recv()
                wo_acc(h)
            wait_small(d0l_ref, 0)
            wait_small(edge_ref, 1)
            wait_small(glob_ref.at[0], 2)
            wait_small(glob_ref.at[2], 4)

        @pl.when(my == 3)
        def _dev3():
            for h in range(HQ):
                ctx_g, l_g = glob_partial(h)
                glob_ref[2, h] = ctx_g.astype(BF16)
                glob_ref[2, 8, :, pl.ds(h, 1)] = l_g.astype(BF16)
            send_small(glob_ref.at[2], 0, (0, 1, 2), 4)
            zero_mid()
            for h in range(HQ):
                copy(p0_ref.at[h, 1], p0_ref.at[h, 1], sp0.at[1, h],
                     rp0.at[1, h], 0).wait_recv()
                copy(p0_ref.at[h, 1], p0_ref.at[h, 1], sp0.at[0, h],
                     rp0.at[1, h], 2).start()
                copy(p0_ref.at[h, 0], p0_ref.at[h, 0], sp0.at[1, h],
                     rp0.at[0, h], 0).wait_recv()
                wo_acc(h)
            wait_small(d0l_ref, 0)
            wait_small(edge_ref, 1)
            wait_small(glob_ref.at[0], 2)
            wait_small(glob_ref.at[1], 3)

        for h in range(HQ):
            hs = pl.ds(h * DH, DH)
            acc_ref[pl.ds(0, 32), hs] = (p0_ref[h, 0, pl.ds(0, 32)].astype(F32)
                                         + glob_ref[0, h].astype(F32)
                                         + glob_ref[1, h].astype(F32)
                                         + glob_ref[2, h].astype(F32))
            acc_ref[pl.ds(32, 128), hs] = (
                p0_ref[h, 1, pl.ds(384, 128)].astype(F32)
                + edge_ref[h].astype(F32))
        lt_ref[pl.ds(0, 32), :] = (d0l_ref[pl.ds(0, 32), :]
                                   + glob_ref[0, 8, :, 0:8].astype(F32)
                                   + glob_ref[1, 8, :, 0:8].astype(F32)
                                   + glob_ref[2, 8, :, 0:8].astype(F32))
        lt_ref[pl.ds(32, 128), :] = (d0l_ref[pl.ds(32, 128), :]
                                     + edge_ref[8, :, 0:8].astype(F32))
        for h in range(HQ):
            hs = pl.ds(h * DH, DH)
            acc_ref[pl.ds(0, 32), hs] = (acc_ref[pl.ds(0, 32), hs]
                                         / lt_ref[pl.ds(0, 32), pl.ds(h, 1)])
            acc_ref[pl.ds(32, 128), hs] = (acc_ref[pl.ds(32, 128), hs]
                                           / lt_ref[pl.ds(32, 128),
                                                    pl.ds(h, 1)])
        out_ref[0, pl.ds(0, 32), :] = jnp.dot(
            acc_ref[pl.ds(0, 32), :], wo_ref[...],
            preferred_element_type=F32)
        out_ref[0, pl.ds(896, 128), :] = jnp.dot(
            acc_ref[pl.ds(32, 128), :], wo_ref[...],
            preferred_element_type=F32)

        @pl.when(my == 0)
        def _():
            for h in range(HQ):
                for j in range(2):
                    copy(p0_ref.at[h, 0], p0_ref.at[h, 0], sp0.at[2 * j, h],
                         rp0.at[0, h], 1).wait_send()
                    copy(p0_ref.at[h, 1], p0_ref.at[h, 1],
                         sp0.at[2 * j + 1, h], rp0.at[1, h], 1).wait_send()
            for j in range(3):
                copy(d0l_ref, d0l_ref, s_small.at[j], r_small.at[0],
                     1).wait_send()

        @pl.when(my == 1)
        def _():
            for h in range(HQ):
                copy(p0_ref.at[h, 0], p0_ref.at[h, 0], sp0.at[0, h],
                     rp0.at[0, h], 2).wait_send()
            for j in range(3):
                copy(edge_ref, edge_ref, s_small.at[j], r_small.at[1],
                     0).wait_send()
                copy(glob_ref.at[0], glob_ref.at[0], s_small.at[3 + j],
                     r_small.at[2], 0).wait_send()

        @pl.when(my == 2)
        def _():
            for j in range(3):
                copy(glob_ref.at[1], glob_ref.at[1], s_small.at[j],
                     r_small.at[3], 0).wait_send()

        @pl.when(my == 3)
        def _():
            for h in range(HQ):
                copy(p0_ref.at[h, 1], p0_ref.at[h, 1], sp0.at[0, h],
                     rp0.at[1, h], 2).wait_send()
            for j in range(3):
                copy(glob_ref.at[2], glob_ref.at[2], s_small.at[j],
                     r_small.at[4], 0).wait_send()

    return pl.pallas_call(
        body,
        out_shape=jax.ShapeDtypeStruct((1, SQ, D), F32),
        in_specs=[pl.BlockSpec(memory_space=pltpu.MemorySpace.VMEM)] * 5,
        out_specs=pl.BlockSpec(memory_space=pltpu.MemorySpace.VMEM),
        scratch_shapes=[
            pltpu.VMEM((HQ, 2, 512, DH), BF16),
            pltpu.VMEM((160, HQ), F32),
            pltpu.VMEM((HQ + 1, 128, DH), BF16),
            pltpu.VMEM((3, HQ + 1, 32, DH), BF16),
            pltpu.VMEM((160, D), F32),
            pltpu.VMEM((160, HQ), F32),
            pltpu.VMEM((4, 256, 512), F32),
            pltpu.SemaphoreType.DMA((4, HQ)),
            pltpu.SemaphoreType.DMA((2, HQ)),
            pltpu.SemaphoreType.DMA((6,)),
            pltpu.SemaphoreType.DMA((5,)),
        ],
        compiler_params=pltpu.CompilerParams(collective_id=0),
    )(x, Wq, K, V, Wo)


# device time: 53691 ns/iter; 1.2000x vs baseline; 1.0378x over previous
import jax
import jax.numpy as jnp
from jax import lax
from jax.experimental import pallas as pl
from jax.experimental.pallas import tpu as pltpu

N_DEV = 4
SQ = 1024
SKV_LOC = 1024
HQ = 8
DH = 128
D = 1024
SCALE = 0.08838834764831843 * 1.4426950408889634

F32 = jnp.float32
BF16 = jnp.bfloat16
MESH = pl.DeviceIdType.MESH


def kernel(x, Wq, K_ext, V_ext, Wo):
    K = jnp.transpose(K_ext[0], (1, 0, 2)).astype(BF16)
    V = jnp.transpose(V_ext[0], (1, 0, 2))

    def body(x_ref, wq_ref, k_ref, v_ref, wo_ref, out_ref,
             p0_ref, d0l_ref, edge_ref, glob_ref, acc_ref, lt_ref,
             madd_ref, sp0, rp0, s_small, r_small):
        my = lax.axis_index("i")

        barrier = pltpu.get_barrier_semaphore()
        for t in range(N_DEV):
            @pl.when(my != t)
            def _():
                pl.semaphore_signal(barrier, inc=1, device_id=(t,),
                                    device_id_type=MESH)
        pl.semaphore_wait(barrier, N_DEV - 1)

        def copy(src, dst, ssem, rsem, dev):
            return pltpu.make_async_remote_copy(
                src_ref=src, dst_ref=dst, send_sem=ssem, recv_sem=rsem,
                device_id=(dev,), device_id_type=MESH)

        def send_small(buf, base, targets, ridx):
            for j, t in enumerate(targets):
                copy(buf, buf, s_small.at[base + j], r_small.at[ridx],
                     t).start()

        def wait_small(buf, ridx):
            copy(buf, buf, s_small.at[0], r_small.at[ridx], 0).wait_recv()

        def q_head(rows, h):
            return (jnp.dot(x_ref[0, rows, :], wq_ref[:, pl.ds(h * DH, DH)],
                            preferred_element_type=F32)
                    * SCALE).astype(BF16)

        def kv(h):
            return k_ref[h], v_ref[h]

        def glob_partial(h):
            k_h, v_h = kv(h)
            qg = q_head(pl.ds(0, 32), h)
            sg = lax.dot_general(qg, k_h, (((1,), (1,)), ((), ())),
                                 preferred_element_type=F32)
            wg = jnp.exp2(sg)
            return (jnp.dot(wg, v_h, preferred_element_type=F32),
                    jnp.sum(wg, axis=1, keepdims=True))

        def zero_mid():
            out_ref[0, pl.ds(32, 864), :] = jnp.zeros((864, D), F32)

        def wo_acc(h):
            hs = pl.ds(h * DH, DH)
            out_ref[0, pl.ds(32, 480), :] = (
                out_ref[0, pl.ds(32, 480), :]
                + jnp.dot(p0_ref[h, 0, pl.ds(32, 480)].astype(F32),
                          wo_ref[hs, :], preferred_element_type=F32))
            out_ref[0, pl.ds(512, 384), :] = (
                out_ref[0, pl.ds(512, 384), :]
                + jnp.dot(p0_ref[h, 1, pl.ds(0, 384)].astype(F32),
                          wo_ref[hs, :], preferred_element_type=F32))

        STARTS = (0, 128, 384, 512)
        @pl.when(my == 0)
        def _dev0():
            for b in range(4):
                qi = lax.broadcasted_iota(jnp.int32, (256, 512), 0) + 256 * b
                kj = lax.broadcasted_iota(jnp.int32, (256, 512), 1) + STARTS[b]
                m = jnp.abs(qi - kj) <= 128
                if b == 0:
                    m = m | (kj < 32) | (qi < 32)
                madd_ref[b] = jnp.where(m, 0.0, -1e9)
            for h in range(HQ):
                q_h = q_head(slice(None), h)
                for b in range(4):
                    st = STARTS[b]
                    qb = q_h[b * 256:(b + 1) * 256]
                    sb = lax.dot_general(qb, k_ref[h, pl.ds(st, 512)],
                                         (((1,), (1,)), ((), ())),
                                         preferred_element_type=F32)
                    wb = jnp.exp2(sb + madd_ref[b])
                    lb = jnp.sum(wb, axis=1, keepdims=True)
                    ctxb = jnp.dot(wb, v_ref[h, pl.ds(st, 512)],
                                   preferred_element_type=F32)
                    if b > 0:
                        st_g = lax.dot_general(qb, k_ref[h, pl.ds(0, 32)],
                                               (((1,), (1,)), ((), ())),
                                               preferred_element_type=F32)
                        wt = jnp.exp2(st_g)
                        lb = lb + jnp.sum(wt, axis=1, keepdims=True)
                        ctxb = ctxb + jnp.dot(wt, v_ref[h, pl.ds(0, 32)],
                                              preferred_element_type=F32)
                    cn = (ctxb / lb).astype(BF16)
                    p0_ref[h, b // 2, pl.ds((b % 2) * 256, 256)] = cn
                    if b == 0:
                        sc = lax.dot_general(q_h[0:32],
                                             k_ref[h, pl.ds(512, 512)],
                                             (((1,), (1,)), ((), ())),
                                             preferred_element_type=F32)
                        wc = jnp.exp2(sc)
                        ctx32 = ctxb[0:32] + jnp.dot(
                            wc, v_ref[h, pl.ds(512, 512)],
                            preferred_element_type=F32)
                        l32 = lb[0:32] + jnp.sum(wc, axis=1, keepdims=True)
                        p0_ref[h, 0, pl.ds(0, 32)] = ctx32.astype(BF16)
                        d0l_ref[pl.ds(0, 32), pl.ds(h, 1)] = l32
                    if b == 3:
                        p0_ref[h, 1, pl.ds(384, 128)] = (
                            ctxb[128:256].astype(BF16))
                        d0l_ref[pl.ds(32, 128), pl.ds(h, 1)] = lb[128:256]
                for j, t in enumerate((1, 3)):
                    copy(p0_ref.at[h], p0_ref.at[h], sp0.at[j, h],
                         rp0.at[0, h], t).start()
            send_small(d0l_ref, 0, (1, 2, 3), 0)
            zero_mid()
            for h in range(HQ):
                wo_acc(h)
            for ridx in (1, 2, 3, 4):
                wait_small(edge_ref if ridx == 1 else
                           glob_ref.at[ridx - 2], ridx)

        @pl.when(my == 1)
        def _dev1():
            qi = lax.broadcasted_iota(jnp.int32, (128, SKV_LOC), 0) + 896
            kj = lax.broadcasted_iota(jnp.int32, (128, SKV_LOC), 1) + SKV_LOC
            madd_e = jnp.where(jnp.abs(qi - kj) <= 128, 0.0, -1e9)
            for h in range(HQ):
                k_h, v_h = kv(h)
                ctx_g, l_g = glob_partial(h)
                glob_ref[0, h] = ctx_g.astype(BF16)
                glob_ref[0, 8, :, pl.ds(h, 1)] = l_g.astype(BF16)
                qe = q_head(pl.ds(896, 128), h)
                se = lax.dot_general(qe, k_h, (((1,), (1,)), ((), ())),
                                     preferred_element_type=F32)
                we = jnp.exp2(se + madd_e)
                edge_ref[h] = jnp.dot(we, v_h,
                                      preferred_element_type=F32).astype(BF16)
                edge_ref[8, :, pl.ds(h, 1)] = jnp.sum(
                    we, axis=1, keepdims=True).astype(BF16)
            send_small(edge_ref, 0, (0, 2, 3), 1)
            send_small(glob_ref.at[0], 3, (0, 2, 3), 2)
            zero_mid()
            for h in range(HQ):
                copy(p0_ref.at[h], p0_ref.at[h], sp0.at[1, h],
                     rp0.at[0, h], 0).wait_recv()
                copy(p0_ref.at[h, 0], p0_ref.at[h, 0], sp0.at[0, h],
                     rp0.at[0, h], 2).start()
                wo_acc(h)
            wait_small(d0l_ref, 0)
            wait_small(glob_ref.at[1], 3)
            wait_small(glob_ref.at[2], 4)

        @pl.when(my == 2)
        def _dev2():
            for h in range(HQ):
                ctx_g, l_g = glob_partial(h)
                glob_ref[1, h] = ctx_g.astype(BF16)
                glob_ref[1, 8, :, pl.ds(h, 1)] = l_g.astype(BF16)
            send_small(glob_ref.at[1], 0, (0, 1, 3), 3)
            zero_mid()
            for h in range(HQ):
                copy(p0_ref.at[h, 0], p0_ref.at[h, 0], sp0.at[1, h],
                     rp0.at[0, h], 0).wait_recv()
                copy(p0_ref.at[h, 1], p0_ref.at[h, 1], sp0.at[1, h],
                     rp0.at[1, h], 0).wait_recv()
                wo_acc(h)
            wait_small(d0l_ref, 0)
            wait_small(edge_ref, 1)
            wait_small(glob_ref.at[0], 2)
            wait_small(glob_ref.at[2], 4)

        @pl.when(my == 3)
        def _dev3():
            for h in range(HQ):
                ctx_g, l_g = glob_partial(h)
                glob_ref[2, h] = ctx_g.astype(BF16)
                glob_ref[2, 8, :, pl.ds(h, 1)] = l_g.astype(BF16)
            send_small(glob_ref.at[2], 0, (0, 1, 2), 4)
            zero_mid()
            for h in range(HQ):
                copy(p0_ref.at[h], p0_ref.at[h], sp0.at[1, h],
                     rp0.at[0, h], 0).wait_recv()
                copy(p0_ref.at[h, 1], p0_ref.at[h, 1], sp0.at[0, h],
                     rp0.at[1, h], 2).start()
                wo_acc(h)
            wait_small(d0l_ref, 0)
            wait_small(edge_ref, 1)
            wait_small(glob_ref.at[0], 2)
            wait_small(glob_ref.at[1], 3)

        for h in range(HQ):
            hs = pl.ds(h * DH, DH)
            acc_ref[pl.ds(0, 32), hs] = (p0_ref[h, 0, pl.ds(0, 32)].astype(F32)
                                         + glob_ref[0, h].astype(F32)
                                         + glob_ref[1, h].astype(F32)
                                         + glob_ref[2, h].astype(F32))
            acc_ref[pl.ds(32, 128), hs] = (
                p0_ref[h, 1, pl.ds(384, 128)].astype(F32)
                + edge_ref[h].astype(F32))
        lt_ref[pl.ds(0, 32), :] = (d0l_ref[pl.ds(0, 32), :]
                                   + glob_ref[0, 8, :, 0:8].astype(F32)
                                   + glob_ref[1, 8, :, 0:8].astype(F32)
                                   + glob_ref[2, 8, :, 0:8].astype(F32))
        lt_ref[pl.ds(32, 128), :] = (d0l_ref[pl.ds(32, 128), :]
                                     + edge_ref[8, :, 0:8].astype(F32))
        for h in range(HQ):
            hs = pl.ds(h * DH, DH)
            acc_ref[pl.ds(0, 32), hs] = (acc_ref[pl.ds(0, 32), hs]
                                         / lt_ref[pl.ds(0, 32), pl.ds(h, 1)])
            acc_ref[pl.ds(32, 128), hs] = (acc_ref[pl.ds(32, 128), hs]
                                           / lt_ref[pl.ds(32, 128),
                                                    pl.ds(h, 1)])
        out_ref[0, pl.ds(0, 32), :] = jnp.dot(
            acc_ref[pl.ds(0, 32), :], wo_ref[...],
            preferred_element_type=F32)
        out_ref[0, pl.ds(896, 128), :] = jnp.dot(
            acc_ref[pl.ds(32, 128), :], wo_ref[...],
            preferred_element_type=F32)

        @pl.when(my == 0)
        def _():
            for h in range(HQ):
                for j in range(2):
                    copy(p0_ref.at[h], p0_ref.at[h], sp0.at[j, h],
                         rp0.at[0, h], 1).wait_send()
            for j in range(3):
                copy(d0l_ref, d0l_ref, s_small.at[j], r_small.at[0],
                     1).wait_send()

        @pl.when(my == 1)
        def _():
            for h in range(HQ):
                copy(p0_ref.at[h, 0], p0_ref.at[h, 0], sp0.at[0, h],
                     rp0.at[0, h], 2).wait_send()
            for j in range(3):
                copy(edge_ref, edge_ref, s_small.at[j], r_small.at[1],
                     0).wait_send()
                copy(glob_ref.at[0], glob_ref.at[0], s_small.at[3 + j],
                     r_small.at[2], 0).wait_send()

        @pl.when(my == 2)
        def _():
            for j in range(3):
                copy(glob_ref.at[1], glob_ref.at[1], s_small.at[j],
                     r_small.at[3], 0).wait_send()

        @pl.when(my == 3)
        def _():
            for h in range(HQ):
                copy(p0_ref.at[h, 1], p0_ref.at[h, 1], sp0.at[0, h],
                     rp0.at[1, h], 2).wait_send()
            for j in range(3):
                copy(glob_ref.at[2], glob_ref.at[2], s_small.at[j],
                     r_small.at[4], 0).wait_send()

    return pl.pallas_call(
        body,
        out_shape=jax.ShapeDtypeStruct((1, SQ, D), F32),
        in_specs=[pl.BlockSpec(memory_space=pltpu.VMEM)] * 5,
        out_specs=pl.BlockSpec(memory_space=pltpu.VMEM),
        scratch_shapes=[
            pltpu.VMEM((HQ, 2, 512, DH), BF16),
            pltpu.VMEM((160, HQ), F32),
            pltpu.VMEM((HQ + 1, 128, DH), BF16),
            pltpu.VMEM((3, HQ + 1, 32, DH), BF16),
            pltpu.VMEM((160, D), F32),
            pltpu.VMEM((160, HQ), F32),
            pltpu.VMEM((4, 256, 512), F32),
            pltpu.SemaphoreType.DMA((2, HQ)),
            pltpu.SemaphoreType.DMA((2, HQ)),
            pltpu.SemaphoreType.DMA((6,)),
            pltpu.SemaphoreType.DMA((5,)),
        ],
        compiler_params=pltpu.CompilerParams(collective_id=0),
    )(x, Wq, K, V, Wo)
